# baseline (speedup 1.0000x reference)
import math

import jax
import jax.numpy as jnp
import numpy as np
from jax.experimental import pallas as pl
from jax.experimental.pallas import tpu as pltpu

_D_MODEL = 768
_GLOVE_DIM = 300
_SEQ = 200
_TI = 8
_NQ = 8


def _pe_const():
    position = np.arange(0, _SEQ, dtype=np.float32)[:, None]
    div_term = np.exp(
        np.arange(0, _D_MODEL, 2, dtype=np.float32)
        * (-math.log(10000.0) / _D_MODEL)
    )
    pe = np.zeros((_SEQ, _D_MODEL), dtype=np.float32)
    pe[:, 0::2] = np.sin(position * div_term)
    pe[:, 1::2] = np.cos(position * div_term)
    return pe


_PE = _pe_const()


_N_DIRECT = 104
_N_HBM = 96
_N_CMEM = 0


def _fused_body(
    toks_ref, glove_hbm, w_ref, b_ref, pe_ref, out_ref, stg_hbm,
    vec_ref, y_ref, sem,
):
    i = pl.program_id(0)

    @pl.when(i == 0)
    def _():
        copies = []
        for j in range(_SEQ):
            src = glove_hbm.at[pl.ds(toks_ref[0, j], 1)]
            if j < _N_DIRECT:
                copies.append(
                    pltpu.make_async_copy(src, vec_ref.at[pl.ds(j, 1)], sem.at[0])
                )
            else:
                copies.append(
                    pltpu.make_async_copy(
                        src, stg_hbm.at[pl.ds(j - _N_DIRECT, 1)], sem.at[1]
                    )
                )
        for c in copies:
            c.start()
        for c in copies:
            c.wait()
        fin = pltpu.make_async_copy(
            stg_hbm, vec_ref.at[pl.ds(_N_DIRECT, _N_HBM)], sem.at[1]
        )
        fin.start()
        fin.wait()
        y_ref[...] = (
            jnp.dot(vec_ref[...], w_ref[...], preferred_element_type=jnp.float32)
            + b_ref[...]
        )

    out_ref[...] = y_ref[...][None, :, :] + pe_ref[...][:, None, :]


@jax.jit
def kernel(tokens, glove_table, W, b):
    S = _SEQ

    pe = jnp.asarray(_PE)
    b2 = b.reshape(1, _D_MODEL)
    toks2 = tokens.reshape(1, S)

    out = pl.pallas_call(
        _fused_body,
        grid=(S // _TI,),
        in_specs=[
            pl.BlockSpec(memory_space=pltpu.SMEM),
            pl.BlockSpec(memory_space=pltpu.HBM),
            pl.BlockSpec((_GLOVE_DIM, _D_MODEL), lambda i: (0, 0)),
            pl.BlockSpec((1, _D_MODEL), lambda i: (0, 0)),
            pl.BlockSpec((_TI, _D_MODEL), lambda i: (i, 0)),
        ],
        out_specs=[
            pl.BlockSpec((_TI, S, _D_MODEL), lambda i: (i, 0, 0)),
            pl.BlockSpec(memory_space=pltpu.HBM),
        ],
        out_shape=[
            jax.ShapeDtypeStruct((S, S, _D_MODEL), jnp.float32),
            jax.ShapeDtypeStruct((_N_HBM, _GLOVE_DIM), jnp.float32),
        ],
        scratch_shapes=[
            pltpu.VMEM((S, _GLOVE_DIM), jnp.float32),
            pltpu.VMEM((S, _D_MODEL), jnp.float32),
            pltpu.SemaphoreType.DMA((2,)),
        ],
    )(toks2, glove_table, W, b2, pe)

    return out[0]
